# baseline (speedup 1.0000x reference)
import jax
import jax.numpy as jnp
from jax.experimental import pallas as pl
from jax.experimental.pallas import tpu as pltpu

_E = 8
_D = 2048
_H = 2048
_B = 2048
_K_TOP = 32

_HT = 512
_BT = 256
_NB = _B // _BT


def _gate_w(gw, gb):
    a_mean = (_D + _K_TOP) / float(_D)
    t_mean = 1.0
    logits = t_mean * gw[0:1, :] + a_mean * gw[1:2, :] + gb
    m = jnp.max(logits)
    p = jnp.exp(logits - m)
    return p / jnp.sum(p)


def _body(gw_ref, gb_ref, experts_ref, x_ref, out_ref,
          acc_ref, cmb_ref, xn_ref):
    h = pl.program_id(0)
    j = pl.program_id(1)

    @pl.when(j < _E)
    def _combine():
        w = _gate_w(gw_ref[...], gb_ref[...])
        idx = jax.lax.broadcasted_iota(jnp.int32, (1, _E), 1)
        we = jnp.sum(jnp.where(idx == j, w, 0.0))
        blk = experts_ref[0] * we

        @pl.when(j == 0)
        def _():
            acc_ref[...] = blk

        @pl.when(j > 0)
        def _():
            acc_ref[...] = acc_ref[...] + blk

        @pl.when(j == _E - 1)
        def _():
            cmb_ref[...] = acc_ref[...].astype(jnp.bfloat16)

    @pl.when(j >= _E)
    def _matmul():
        bt = j - _E

        @pl.when(h == 0)
        def _():
            x = x_ref[...]
            mean = jnp.mean(x, axis=-1, keepdims=True)
            cen = x - mean
            std = jnp.sqrt(jnp.mean(cen * cen, axis=-1, keepdims=True))
            xn_ref[pl.ds(bt * _BT, _BT), :] = (
                cen / (std + 1e-6)).astype(jnp.bfloat16)

        out_ref[...] = jnp.dot(xn_ref[pl.ds(bt * _BT, _BT), :], cmb_ref[...],
                               preferred_element_type=jnp.float32)


def kernel(query_embedding, gate_W, gate_b, experts):
    gb2 = gate_b.reshape(1, _E)

    def _e_idx(h, j):
        return (jnp.minimum(j, _E - 1), 0, h)

    def _b_idx(h, j):
        bt = jnp.clip(j - _E, 0, _NB - 1)
        return (bt, 0)

    def _o_idx(h, j):
        bt = jnp.clip(j - _E, 0, _NB - 1)
        return (bt, h)

    out = pl.pallas_call(
        _body,
        grid=(_H // _HT, _E + _NB),
        in_specs=[
            pl.BlockSpec((12, _E), lambda h, j: (0, 0)),
            pl.BlockSpec((1, _E), lambda h, j: (0, 0)),
            pl.BlockSpec((1, _D, _HT), _e_idx),
            pl.BlockSpec((_BT, _D), _b_idx),
        ],
        out_specs=pl.BlockSpec((_BT, _HT), _o_idx),
        out_shape=jax.ShapeDtypeStruct((_B, _H), jnp.float32),
        scratch_shapes=[
            pltpu.VMEM((_D, _HT), jnp.float32),
            pltpu.VMEM((_D, _HT), jnp.bfloat16),
            pltpu.VMEM((_B, _D), jnp.bfloat16),
        ],
    )(gate_W, gb2, experts, query_embedding)

    return out


# speedup vs baseline: 1.3421x; 1.3421x over previous
import jax
import jax.numpy as jnp
from jax.experimental import pallas as pl
from jax.experimental.pallas import tpu as pltpu

_E = 8
_D = 2048
_H = 2048
_B = 2048
_K_TOP = 32

_DT = 128
_HT = 512


def _gate_w(gw, gb):
    a_mean = (_D + _K_TOP) / float(_D)
    t_mean = 1.0
    logits = t_mean * gw[0:1, :] + a_mean * gw[1:2, :] + gb
    m = jnp.max(logits)
    p = jnp.exp(logits - m)
    return p / jnp.sum(p)


def _combine_body(gw_ref, gb_ref, experts_ref, out_ref):
    w = _gate_w(gw_ref[...], gb_ref[...])
    blk = experts_ref[...]
    acc = blk[0] * w[0, 0]
    for e in range(1, _E):
        acc = acc + blk[e] * w[0, e]
    out_ref[...] = acc.astype(jnp.bfloat16)


def _matmul_body(x_ref, c_ref, out_ref, xn_ref):
    h = pl.program_id(0)

    @pl.when(h == 0)
    def _():
        x = x_ref[...]
        mean = jnp.mean(x, axis=-1, keepdims=True)
        cen = x - mean
        std = jnp.sqrt(jnp.mean(cen * cen, axis=-1, keepdims=True))
        xn_ref[...] = (cen / (std + 1e-6)).astype(jnp.bfloat16)

    out_ref[...] = jnp.dot(xn_ref[...], c_ref[...],
                           preferred_element_type=jnp.float32)


def kernel(query_embedding, gate_W, gate_b, experts):
    gb2 = gate_b.reshape(1, _E)

    combined = pl.pallas_call(
        _combine_body,
        grid=(_D // _DT,),
        in_specs=[
            pl.BlockSpec((12, _E), lambda d: (0, 0)),
            pl.BlockSpec((1, _E), lambda d: (0, 0)),
            pl.BlockSpec((_E, _DT, _H), lambda d: (0, d, 0)),
        ],
        out_specs=pl.BlockSpec((_DT, _H), lambda d: (d, 0)),
        out_shape=jax.ShapeDtypeStruct((_D, _H), jnp.bfloat16),
    )(gate_W, gb2, experts)

    out = pl.pallas_call(
        _matmul_body,
        grid=(_H // _HT,),
        in_specs=[
            pl.BlockSpec((_B, _D), lambda h: (0, 0)),
            pl.BlockSpec((_D, _HT), lambda h: (0, h)),
        ],
        out_specs=pl.BlockSpec((_B, _HT), lambda h: (0, h)),
        out_shape=jax.ShapeDtypeStruct((_B, _H), jnp.float32),
        scratch_shapes=[pltpu.VMEM((_B, _D), jnp.bfloat16)],
    )(query_embedding, combined)

    return out
